# baseline (speedup 1.0000x reference)
import jax
import jax.numpy as jnp
from jax.experimental import pallas as pl
from jax.experimental.pallas import tpu as pltpu

_SUB = 128


def _tc_pool_block(a_ref, x_ref, o_ref):
    L = x_ref.shape[0] // o_ref.shape[0]
    for t in range(o_ref.shape[0] // _SUB):
        o_ref[t * _SUB:(t + 1) * _SUB, :] = jax.lax.dot(
            a_ref[...],
            x_ref[t * _SUB * L:(t + 1) * _SUB * L, :],
            preferred_element_type=jnp.float32,
        )


def kernel(hidden_states, word_boundaries):
    B, S, D = hidden_states.shape
    W = word_boundaries.shape[1]
    L = S // W
    R = B * W
    x = hidden_states.reshape(B * S, D)
    blk = 1024
    row = jax.lax.broadcasted_iota(jnp.int32, (_SUB, _SUB * L), 0)
    col = jax.lax.broadcasted_iota(jnp.int32, (_SUB, _SUB * L), 1)
    pool_mat = jnp.where(col // L == row, 1.0 / L, 0.0).astype(hidden_states.dtype)
    out = pl.pallas_call(
        _tc_pool_block,
        grid=(R // blk,),
        in_specs=[
            pl.BlockSpec((_SUB, _SUB * L), lambda i: (0, 0)),
            pl.BlockSpec((blk * L, D), lambda i: (i, 0)),
        ],
        out_specs=pl.BlockSpec((blk, D), lambda i: (i, 0)),
        out_shape=jax.ShapeDtypeStruct((R, D), hidden_states.dtype),
        compiler_params=pltpu.CompilerParams(
            dimension_semantics=("arbitrary",),
        ),
    )(pool_mat, x)
    return out


# speedup vs baseline: 1.0080x; 1.0080x over previous
import jax
import jax.numpy as jnp
from jax.experimental import pallas as pl
from jax.experimental.pallas import tpu as pltpu

_SUB = 64


def _tc_pool_block(a_ref, x_ref, o_ref):
    L = x_ref.shape[0] // o_ref.shape[0]
    for t in range(o_ref.shape[0] // _SUB):
        o_ref[t * _SUB:(t + 1) * _SUB, :] = jax.lax.dot(
            a_ref[...],
            x_ref[t * _SUB * L:(t + 1) * _SUB * L, :],
            preferred_element_type=jnp.float32,
        )


def kernel(hidden_states, word_boundaries):
    B, S, D = hidden_states.shape
    W = word_boundaries.shape[1]
    L = S // W
    R = B * W
    x = hidden_states.reshape(B * S, D)
    blk = 512
    row = jax.lax.broadcasted_iota(jnp.int32, (_SUB, _SUB * L), 0)
    col = jax.lax.broadcasted_iota(jnp.int32, (_SUB, _SUB * L), 1)
    pool_mat = jnp.where(col // L == row, 1.0 / L, 0.0).astype(hidden_states.dtype)
    out = pl.pallas_call(
        _tc_pool_block,
        grid=(R // blk,),
        in_specs=[
            pl.BlockSpec((_SUB, _SUB * L), lambda i: (0, 0)),
            pl.BlockSpec((blk * L, D), lambda i: (i, 0)),
        ],
        out_specs=pl.BlockSpec((blk, D), lambda i: (i, 0)),
        out_shape=jax.ShapeDtypeStruct((R, D), hidden_states.dtype),
        compiler_params=pltpu.CompilerParams(
            dimension_semantics=("arbitrary",),
        ),
    )(pool_mat, x)
    return out
